# baseline (speedup 1.0000x reference)
import functools

import jax
import jax.numpy as jnp
from jax import lax
from jax.experimental import pallas as pl
from jax.experimental.pallas import tpu as pltpu
from jax.experimental.pallas import tpu_sc as plsc

N = 10000
E = 320000
D_IN = 128
HID = 128
OUT = 128
CAT1 = 384
EPS = 1e-5

NC = 2
NS = 16
NW = NC * NS
L = 16

EPW = 10240
EPAD = NW * EPW
KB = 40
NB = EPW // KB
CHB = 32
NCHUNK = NB // CHB
PAIRS = CHB // 2
NP = 10240
RPT = NP // NS

RB = 2000
GRID = N // RB


def _sc_mesh():
    return plsc.VectorSubcoreMesh(core_axis_name="c", subcore_axis_name="s")


def _deg_parts(col3):
    @functools.partial(
        pl.kernel,
        out_type=jax.ShapeDtypeStruct((NC, NP, L), jnp.float32),
        mesh=_sc_mesh(),
        scratch_types=[
            pltpu.VMEM_SHARED((NP, L), jnp.float32),
            pltpu.VMEM((NB, KB), jnp.int32),
            pltpu.VMEM((KB, L), jnp.float32),
            pltpu.VMEM((RPT, L), jnp.float32),
        ],
    )
    def k(col_hbm, out_hbm, acc, cv, ov, zv):
        c = lax.axis_index("c")
        s = lax.axis_index("s")
        wid = s * NC + c

        def fill(i, cy):
            zv[i, :] = jnp.zeros((L,), jnp.float32)
            return cy

        lax.fori_loop(0, RPT, fill, jnp.int32(0))

        def fill1(i, cy):
            ov[i, :] = jnp.ones((L,), jnp.float32)
            return cy

        lax.fori_loop(0, KB, fill1, jnp.int32(0))

        pltpu.sync_copy(zv, acc.at[pl.ds(s * RPT, RPT)])
        pltpu.sync_copy(col_hbm.at[wid], cv)
        plsc.subcore_barrier()

        def body(g, cy):
            pltpu.sync_copy(ov, acc.at[cv.at[g]], add=True)
            return cy

        lax.fori_loop(0, NB, body, jnp.int32(0))

        plsc.subcore_barrier()
        pltpu.sync_copy(acc.at[pl.ds(s * RPT, RPT)],
                        out_hbm.at[c, pl.ds(s * RPT, RPT)])

    return k(col3)


def _spmm_parts(y, rc4):
    i0 = jnp.int32(0)
    i1 = jnp.int32(1)

    @functools.partial(
        pl.kernel,
        out_type=jax.ShapeDtypeStruct((NC, NP, HID), jnp.float32),
        mesh=_sc_mesh(),
        scratch_types=[
            pltpu.VMEM_SHARED((NP, HID), jnp.float32),
            pltpu.VMEM((CHB, 2, KB), jnp.int32),
            pltpu.VMEM((KB, HID), jnp.float32),
            pltpu.VMEM((KB, HID), jnp.float32),
            pltpu.SemaphoreType.DMA,
            pltpu.SemaphoreType.DMA,
        ],
    )
    def k(y_hbm, rc_hbm, out_hbm, acc, rcc, gv0, gv1, sg0, sg1):
        c = lax.axis_index("c")
        s = lax.axis_index("s")
        wid = s * NC + c

        def fill(i, cy):
            for j in range(HID // L):
                gv0[i, pl.ds(j * L, L)] = jnp.zeros((L,), jnp.float32)
            return cy

        lax.fori_loop(0, KB, fill, jnp.int32(0))
        for r in range(RPT // KB):
            pltpu.sync_copy(gv0, acc.at[pl.ds(s * RPT + r * KB, KB)])

        plsc.subcore_barrier()

        for ch in range(NCHUNK):
            pltpu.sync_copy(rc_hbm.at[wid, pl.ds(ch * CHB, CHB)], rcc)
            pltpu.async_copy(y_hbm.at[rcc.at[i0, i0]], gv0, sg0)

            def pair(t, cy):
                b0 = t * 2
                b1 = b0 + 1
                pltpu.async_copy(y_hbm.at[rcc.at[b1, i0]], gv1, sg1)
                pltpu.make_async_copy(y_hbm.at[rcc.at[b0, i0]], gv0,
                                      sg0).wait()
                pltpu.sync_copy(gv0, acc.at[rcc.at[b0, i1]], add=True)

                @pl.when(t < PAIRS - 1)
                def _():
                    pltpu.async_copy(y_hbm.at[rcc.at[b0 + 2, i0]], gv0, sg0)

                pltpu.make_async_copy(y_hbm.at[rcc.at[b1, i0]], gv1,
                                      sg1).wait()
                pltpu.sync_copy(gv1, acc.at[rcc.at[b1, i1]], add=True)
                return cy

            lax.fori_loop(jnp.int32(0), jnp.int32(PAIRS), pair, jnp.int32(0))

        plsc.subcore_barrier()
        pltpu.sync_copy(acc.at[pl.ds(s * RPT, RPT)],
                        out_hbm.at[c, pl.ds(s * RPT, RPT)])

    return k(y, rc4)


def _k_in(x, wcat, bcat, d1):
    din = x.shape[1]

    def body(x_ref, w_ref, b_ref, d_ref, o0, o1, o2):
        acc = jnp.dot(x_ref[...], w_ref[...],
                      preferred_element_type=jnp.float32) + b_ref[...]
        dd = d_ref[...]
        o0[...] = acc[:, :HID]
        o1[...] = acc[:, HID:2 * HID] * dd
        o2[...] = acc[:, 2 * HID:] * dd

    return pl.pallas_call(
        body,
        grid=(GRID,),
        in_specs=[
            pl.BlockSpec((RB, din), lambda i: (i, i * 0)),
            pl.BlockSpec((din, 3 * HID), lambda i: (i * 0, i * 0)),
            pl.BlockSpec((1, 3 * HID), lambda i: (i * 0, i * 0)),
            pl.BlockSpec((RB, 1), lambda i: (i, i * 0)),
        ],
        out_specs=[pl.BlockSpec((RB, HID), lambda i: (i, i * 0))] * 3,
        out_shape=[jax.ShapeDtypeStruct((N, HID), jnp.float32)] * 3,
    )(x, wcat, bcat, d1)


def _k_bn_in(h, scale, shift, wcat, bcat, d1):
    def body(h_ref, sc_ref, sh_ref, w_ref, b_ref, d_ref, o0, o1, o2):
        hn = jnp.maximum(h_ref[...] * sc_ref[...] + sh_ref[...], 0.0)
        acc = jnp.dot(hn, w_ref[...],
                      preferred_element_type=jnp.float32) + b_ref[...]
        dd = d_ref[...]
        o0[...] = acc[:, :HID]
        o1[...] = acc[:, HID:2 * HID] * dd
        o2[...] = acc[:, 2 * HID:] * dd

    return pl.pallas_call(
        body,
        grid=(GRID,),
        in_specs=[
            pl.BlockSpec((RB, CAT1), lambda i: (i, i * 0)),
            pl.BlockSpec((1, CAT1), lambda i: (i * 0, i * 0)),
            pl.BlockSpec((1, CAT1), lambda i: (i * 0, i * 0)),
            pl.BlockSpec((CAT1, 3 * HID), lambda i: (i * 0, i * 0)),
            pl.BlockSpec((1, 3 * HID), lambda i: (i * 0, i * 0)),
            pl.BlockSpec((RB, 1), lambda i: (i, i * 0)),
        ],
        out_specs=[pl.BlockSpec((RB, HID), lambda i: (i, i * 0))] * 3,
        out_shape=[jax.ShapeDtypeStruct((N, HID), jnp.float32)] * 3,
    )(h, scale, shift, wcat, bcat, d1)


def _k_combine(p0, p1, y, d2):
    def body(a_ref, b_ref, y_ref, d_ref, o_ref):
        o_ref[...] = d_ref[...] * (a_ref[...] + b_ref[...] + y_ref[...])

    return pl.pallas_call(
        body,
        grid=(GRID,),
        in_specs=[
            pl.BlockSpec((RB, HID), lambda i: (i, i * 0)),
            pl.BlockSpec((RB, HID), lambda i: (i, i * 0)),
            pl.BlockSpec((RB, HID), lambda i: (i, i * 0)),
            pl.BlockSpec((RB, 1), lambda i: (i, i * 0)),
        ],
        out_specs=pl.BlockSpec((RB, HID), lambda i: (i, i * 0)),
        out_shape=jax.ShapeDtypeStruct((N, HID), jnp.float32),
    )(p0, p1, y, d2)


def _k_assemble(x0, a1, s10, s11, y2, s20, s21, d1):
    def body(x0_ref, a1_ref, s10_ref, s11_ref, y2_ref, s20_ref, s21_ref,
             d_ref, h_ref, sm_ref, sq_ref):
        i = pl.program_id(0)
        dd = d_ref[...]
        h1 = dd * (s10_ref[...] + s11_ref[...] + a1_ref[...])
        h2 = dd * (s20_ref[...] + s21_ref[...] + y2_ref[...])
        hb = jnp.concatenate([x0_ref[...], h1, h2], axis=1)
        h_ref[...] = hb
        pm = jnp.broadcast_to(jnp.sum(hb, axis=0, keepdims=True), (8, CAT1))
        pq = jnp.broadcast_to(jnp.sum(hb * hb, axis=0, keepdims=True),
                              (8, CAT1))

        @pl.when(i == 0)
        def _():
            sm_ref[...] = pm
            sq_ref[...] = pq

        @pl.when(i > 0)
        def _():
            sm_ref[...] = sm_ref[...] + pm
            sq_ref[...] = sq_ref[...] + pq

    return pl.pallas_call(
        body,
        grid=(GRID,),
        in_specs=[pl.BlockSpec((RB, HID), lambda i: (i, i * 0))] * 7 +
                 [pl.BlockSpec((RB, 1), lambda i: (i, i * 0))],
        out_specs=[
            pl.BlockSpec((RB, CAT1), lambda i: (i, i * 0)),
            pl.BlockSpec((8, CAT1), lambda i: (i * 0, i * 0)),
            pl.BlockSpec((8, CAT1), lambda i: (i * 0, i * 0)),
        ],
        out_shape=[
            jax.ShapeDtypeStruct((N, CAT1), jnp.float32),
            jax.ShapeDtypeStruct((8, CAT1), jnp.float32),
            jax.ShapeDtypeStruct((8, CAT1), jnp.float32),
        ],
    )(x0, a1, s10, s11, y2, s20, s21, d1)


def _k_final(y0, b1p, u10, u11, y2b, v20, v21, d1, wf, bf2):
    def body(y0_ref, b1_ref, u10_ref, u11_ref, y2_ref, v20_ref, v21_ref,
             d_ref, wf_ref, bf_ref, o_ref):
        dd = d_ref[...]
        g1 = dd * (u10_ref[...] + u11_ref[...] + b1_ref[...])
        g2 = dd * (v20_ref[...] + v21_ref[...] + y2_ref[...])
        wf = wf_ref[...]
        acc = jnp.dot(y0_ref[...], wf[:HID], preferred_element_type=jnp.float32)
        acc += jnp.dot(g1, wf[HID:2 * HID], preferred_element_type=jnp.float32)
        acc += jnp.dot(g2, wf[2 * HID:], preferred_element_type=jnp.float32)
        o_ref[...] = acc + bf_ref[...]

    return pl.pallas_call(
        body,
        grid=(GRID,),
        in_specs=[pl.BlockSpec((RB, HID), lambda i: (i, i * 0))] * 7 +
                 [
                     pl.BlockSpec((RB, 1), lambda i: (i, i * 0)),
                     pl.BlockSpec((CAT1, OUT), lambda i: (i * 0, i * 0)),
                     pl.BlockSpec((1, OUT), lambda i: (i * 0, i * 0)),
                 ],
        out_specs=pl.BlockSpec((RB, OUT), lambda i: (i, i * 0)),
        out_shape=jax.ShapeDtypeStruct((N, OUT), jnp.float32),
    )(y0, b1p, u10, u11, y2b, v20, v21, d1, wf, bf2)


def kernel(x, edge_index, W0, b0, W1, b1, gamma, beta, Wf, bf):
    x = x.astype(jnp.float32)
    row = edge_index[0].astype(jnp.int32)
    col = edge_index[1].astype(jnp.int32)
    npad = EPAD - E
    rowp = jnp.concatenate([row, jnp.zeros((npad,), jnp.int32)])
    dummy_col = N + jnp.arange(npad, dtype=jnp.int32) % (NP - N)
    colp = jnp.concatenate([col, dummy_col])
    row3 = rowp.reshape(NW, NB, KB)
    col3 = colp.reshape(NW, NB, KB)
    rc4 = jnp.stack([row3, col3], axis=2)

    degp = _deg_parts(col3)
    deg = degp[0, :N, 0] + degp[1, :N, 0] + 1.0
    dinv = lax.rsqrt(deg)
    d1 = dinv[:, None]
    d2 = (dinv * dinv)[:, None]

    w0cat = jnp.concatenate([W0[0], W0[1], W0[2]], axis=1)
    b0cat = jnp.concatenate([b0[0], b0[1], b0[2]])[None, :]
    x0, a1p, a2p = _k_in(x, w0cat, b0cat, d1)

    s1 = _spmm_parts(a1p, rc4)
    t1 = _spmm_parts(a2p, rc4)
    y2 = _k_combine(t1[0], t1[1], a2p, d2)
    s2 = _spmm_parts(y2, rc4)

    h, sm, sq = _k_assemble(x0, a1p, s1[0], s1[1], y2, s2[0], s2[1], d1)
    ssum = sm[0]
    ssq = sq[0]
    mean = ssum / N
    var = ssq / N - mean * mean
    rstd = lax.rsqrt(var + EPS)
    scale = (gamma * rstd)[None, :]
    shift = (beta - mean * gamma * rstd)[None, :]

    w1cat = jnp.concatenate([W1[0], W1[1], W1[2]], axis=1)
    b1cat = jnp.concatenate([b1[0], b1[1], b1[2]])[None, :]
    y0, b1p, b2p = _k_bn_in(h, scale, shift, w1cat, b1cat, d1)

    u1 = _spmm_parts(b1p, rc4)
    v1 = _spmm_parts(b2p, rc4)
    y2b = _k_combine(v1[0], v1[1], b2p, d2)
    v2 = _spmm_parts(y2b, rc4)

    return _k_final(y0, b1p, u1[0], u1[1], y2b, v2[0], v2[1], d1, Wf,
                    bf[None, :])


# speedup vs baseline: 2.5832x; 2.5832x over previous
import functools

import jax
import jax.numpy as jnp
from jax import lax
from jax.experimental import pallas as pl
from jax.experimental.pallas import tpu as pltpu
from jax.experimental.pallas import tpu_sc as plsc

N = 10000
E = 320000
D_IN = 128
HID = 128
OUT = 128
CAT1 = 384
EPS = 1e-5

NC = 2
NS = 16
NW = NC * NS
L = 16

EPW = E // NW
KB = 40
NB = EPW // KB
CH = 5
BPC = NB // CH
PAIRS = BPC // 2
NP = 10240
RPT = NP // NS

RB = 2000
GRID = N // RB


def _sc_mesh():
    return plsc.VectorSubcoreMesh(core_axis_name="c", subcore_axis_name="s")


KD = 80
NBD = (E // NW) // KD


def _deg_parts(col3):
    @functools.partial(
        pl.kernel,
        out_type=jax.ShapeDtypeStruct((NC, NP, L), jnp.float32),
        mesh=_sc_mesh(),
        scratch_types=[
            pltpu.VMEM_SHARED((NP, L), jnp.float32),
            pltpu.VMEM((NBD, KD), jnp.int32),
            pltpu.VMEM((KD, L), jnp.float32),
            pltpu.VMEM((RPT, L), jnp.float32),
        ],
    )
    def k(col_hbm, out_hbm, acc, cv, ov, zv):
        c = lax.axis_index("c")
        s = lax.axis_index("s")
        wid = s * NC + c

        def fill(i, cy):
            zv[i, :] = jnp.zeros((L,), jnp.float32)
            return cy

        lax.fori_loop(0, RPT, fill, jnp.int32(0))

        def fill1(i, cy):
            ov[i, :] = jnp.ones((L,), jnp.float32)
            return cy

        lax.fori_loop(0, KD, fill1, jnp.int32(0))

        pltpu.sync_copy(zv, acc.at[pl.ds(s * RPT, RPT)])
        pltpu.sync_copy(col_hbm.at[wid], cv)
        plsc.subcore_barrier()

        def body(g, cy):
            pltpu.sync_copy(ov, acc.at[cv.at[g]], add=True)
            return cy

        lax.fori_loop(0, NBD, body, jnp.int32(0))

        plsc.subcore_barrier()
        pltpu.sync_copy(acc.at[pl.ds(s * RPT, RPT)],
                        out_hbm.at[c, pl.ds(s * RPT, RPT)])

    return k(col3)


def _spmm_parts(y, rc4):
    i0 = jnp.int32(0)
    i1 = jnp.int32(1)

    @functools.partial(
        pl.kernel,
        out_type=jax.ShapeDtypeStruct((NC, NP, HID), jnp.float32),
        mesh=_sc_mesh(),
        scratch_types=[
            pltpu.VMEM_SHARED((NP, HID), jnp.float32),
            pltpu.VMEM((BPC, 2, KB), jnp.int32),
            pltpu.VMEM((KB, HID), jnp.float32),
            pltpu.VMEM((KB, HID), jnp.float32),
            pltpu.SemaphoreType.DMA,
            pltpu.SemaphoreType.DMA,
        ],
    )
    def k(y_hbm, rc_hbm, out_hbm, acc, rcc, gv0, gv1, sg0, sg1):
        c = lax.axis_index("c")
        s = lax.axis_index("s")
        wid = s * NC + c

        def fill(i, cy):
            for j in range(HID // L):
                gv0[i, pl.ds(j * L, L)] = jnp.zeros((L,), jnp.float32)
            return cy

        lax.fori_loop(0, KB, fill, jnp.int32(0))
        for r in range(RPT // KB):
            pltpu.sync_copy(gv0, acc.at[pl.ds(s * RPT + r * KB, KB)])

        plsc.subcore_barrier()

        for ch in range(CH):
            pltpu.sync_copy(rc_hbm.at[wid, pl.ds(ch * BPC, BPC)], rcc)
            pltpu.async_copy(y_hbm.at[rcc.at[i0, i0]], gv0, sg0)

            def pair(t, cy):
                b0 = t * 2
                b1 = b0 + 1
                pltpu.async_copy(y_hbm.at[rcc.at[b1, i0]], gv1, sg1)
                pltpu.make_async_copy(y_hbm.at[rcc.at[b0, i0]], gv0,
                                      sg0).wait()
                pltpu.sync_copy(gv0, acc.at[rcc.at[b0, i1]], add=True)

                @pl.when(t < PAIRS - 1)
                def _():
                    pltpu.async_copy(y_hbm.at[rcc.at[b0 + 2, i0]], gv0, sg0)

                pltpu.make_async_copy(y_hbm.at[rcc.at[b1, i0]], gv1,
                                      sg1).wait()
                pltpu.sync_copy(gv1, acc.at[rcc.at[b1, i1]], add=True)
                return cy

            lax.fori_loop(jnp.int32(0), jnp.int32(PAIRS), pair, jnp.int32(0))

        plsc.subcore_barrier()
        pltpu.sync_copy(acc.at[pl.ds(s * RPT, RPT)],
                        out_hbm.at[c, pl.ds(s * RPT, RPT)])

    return k(y, rc4)


def _k_in(x, wcat, bcat, d1):
    din = x.shape[1]

    def body(x_ref, w_ref, b_ref, d_ref, o0, o1, o2):
        acc = jnp.dot(x_ref[...], w_ref[...],
                      preferred_element_type=jnp.float32) + b_ref[...]
        dd = d_ref[...]
        o0[...] = acc[:, :HID]
        o1[...] = acc[:, HID:2 * HID] * dd
        o2[...] = acc[:, 2 * HID:] * dd

    return pl.pallas_call(
        body,
        grid=(GRID,),
        in_specs=[
            pl.BlockSpec((RB, din), lambda i: (i, i * 0)),
            pl.BlockSpec((din, 3 * HID), lambda i: (i * 0, i * 0)),
            pl.BlockSpec((1, 3 * HID), lambda i: (i * 0, i * 0)),
            pl.BlockSpec((RB, 1), lambda i: (i, i * 0)),
        ],
        out_specs=[pl.BlockSpec((RB, HID), lambda i: (i, i * 0))] * 3,
        out_shape=[jax.ShapeDtypeStruct((N, HID), jnp.float32)] * 3,
    )(x, wcat, bcat, d1)


def _k_bn_in(h, scale, shift, wcat, bcat, d1):
    def body(h_ref, sc_ref, sh_ref, w_ref, b_ref, d_ref, o0, o1, o2):
        hn = jnp.maximum(h_ref[...] * sc_ref[...] + sh_ref[...], 0.0)
        acc = jnp.dot(hn, w_ref[...],
                      preferred_element_type=jnp.float32) + b_ref[...]
        dd = d_ref[...]
        o0[...] = acc[:, :HID]
        o1[...] = acc[:, HID:2 * HID] * dd
        o2[...] = acc[:, 2 * HID:] * dd

    return pl.pallas_call(
        body,
        grid=(GRID,),
        in_specs=[
            pl.BlockSpec((RB, CAT1), lambda i: (i, i * 0)),
            pl.BlockSpec((1, CAT1), lambda i: (i * 0, i * 0)),
            pl.BlockSpec((1, CAT1), lambda i: (i * 0, i * 0)),
            pl.BlockSpec((CAT1, 3 * HID), lambda i: (i * 0, i * 0)),
            pl.BlockSpec((1, 3 * HID), lambda i: (i * 0, i * 0)),
            pl.BlockSpec((RB, 1), lambda i: (i, i * 0)),
        ],
        out_specs=[pl.BlockSpec((RB, HID), lambda i: (i, i * 0))] * 3,
        out_shape=[jax.ShapeDtypeStruct((N, HID), jnp.float32)] * 3,
    )(h, scale, shift, wcat, bcat, d1)


def _k_combine(p0, p1, y, d2):
    def body(a_ref, b_ref, y_ref, d_ref, o_ref):
        o_ref[...] = d_ref[...] * (a_ref[...] + b_ref[...] + y_ref[...])

    return pl.pallas_call(
        body,
        grid=(GRID,),
        in_specs=[
            pl.BlockSpec((RB, HID), lambda i: (i, i * 0)),
            pl.BlockSpec((RB, HID), lambda i: (i, i * 0)),
            pl.BlockSpec((RB, HID), lambda i: (i, i * 0)),
            pl.BlockSpec((RB, 1), lambda i: (i, i * 0)),
        ],
        out_specs=pl.BlockSpec((RB, HID), lambda i: (i, i * 0)),
        out_shape=jax.ShapeDtypeStruct((N, HID), jnp.float32),
    )(p0, p1, y, d2)


def _k_assemble(x0, a1, s10, s11, y2, s20, s21, d1):
    def body(x0_ref, a1_ref, s10_ref, s11_ref, y2_ref, s20_ref, s21_ref,
             d_ref, h_ref, sm_ref, sq_ref):
        i = pl.program_id(0)
        dd = d_ref[...]
        h1 = dd * (s10_ref[...] + s11_ref[...] + a1_ref[...])
        h2 = dd * (s20_ref[...] + s21_ref[...] + y2_ref[...])
        hb = jnp.concatenate([x0_ref[...], h1, h2], axis=1)
        h_ref[...] = hb
        pm = jnp.broadcast_to(jnp.sum(hb, axis=0, keepdims=True), (8, CAT1))
        pq = jnp.broadcast_to(jnp.sum(hb * hb, axis=0, keepdims=True),
                              (8, CAT1))

        @pl.when(i == 0)
        def _():
            sm_ref[...] = pm
            sq_ref[...] = pq

        @pl.when(i > 0)
        def _():
            sm_ref[...] = sm_ref[...] + pm
            sq_ref[...] = sq_ref[...] + pq

    return pl.pallas_call(
        body,
        grid=(GRID,),
        in_specs=[pl.BlockSpec((RB, HID), lambda i: (i, i * 0))] * 7 +
                 [pl.BlockSpec((RB, 1), lambda i: (i, i * 0))],
        out_specs=[
            pl.BlockSpec((RB, CAT1), lambda i: (i, i * 0)),
            pl.BlockSpec((8, CAT1), lambda i: (i * 0, i * 0)),
            pl.BlockSpec((8, CAT1), lambda i: (i * 0, i * 0)),
        ],
        out_shape=[
            jax.ShapeDtypeStruct((N, CAT1), jnp.float32),
            jax.ShapeDtypeStruct((8, CAT1), jnp.float32),
            jax.ShapeDtypeStruct((8, CAT1), jnp.float32),
        ],
    )(x0, a1, s10, s11, y2, s20, s21, d1)


def _k_final(y0, b1p, u10, u11, y2b, v20, v21, d1, wf, bf2):
    def body(y0_ref, b1_ref, u10_ref, u11_ref, y2_ref, v20_ref, v21_ref,
             d_ref, wf_ref, bf_ref, o_ref):
        dd = d_ref[...]
        g1 = dd * (u10_ref[...] + u11_ref[...] + b1_ref[...])
        g2 = dd * (v20_ref[...] + v21_ref[...] + y2_ref[...])
        wf = wf_ref[...]
        acc = jnp.dot(y0_ref[...], wf[:HID], preferred_element_type=jnp.float32)
        acc += jnp.dot(g1, wf[HID:2 * HID], preferred_element_type=jnp.float32)
        acc += jnp.dot(g2, wf[2 * HID:], preferred_element_type=jnp.float32)
        o_ref[...] = acc + bf_ref[...]

    return pl.pallas_call(
        body,
        grid=(GRID,),
        in_specs=[pl.BlockSpec((RB, HID), lambda i: (i, i * 0))] * 7 +
                 [
                     pl.BlockSpec((RB, 1), lambda i: (i, i * 0)),
                     pl.BlockSpec((CAT1, OUT), lambda i: (i * 0, i * 0)),
                     pl.BlockSpec((1, OUT), lambda i: (i * 0, i * 0)),
                 ],
        out_specs=pl.BlockSpec((RB, OUT), lambda i: (i, i * 0)),
        out_shape=jax.ShapeDtypeStruct((N, OUT), jnp.float32),
    )(y0, b1p, u10, u11, y2b, v20, v21, d1, wf, bf2)


def kernel(x, edge_index, W0, b0, W1, b1, gamma, beta, Wf, bf):
    x = x.astype(jnp.float32)
    row = edge_index[0].astype(jnp.int32)
    col = edge_index[1].astype(jnp.int32)
    row3 = row.reshape(NW, NB, KB)
    col3 = col.reshape(NW, NB, KB)
    rc4 = jnp.stack([row3, col3], axis=2)
    col3d = col.reshape(NW, NBD, KD)

    degp = _deg_parts(col3d)
    deg = degp[0, :N, 0] + degp[1, :N, 0] + 1.0
    dinv = lax.rsqrt(deg)
    d1 = dinv[:, None]
    d2 = (dinv * dinv)[:, None]

    w0cat = jnp.concatenate([W0[0], W0[1], W0[2]], axis=1)
    b0cat = jnp.concatenate([b0[0], b0[1], b0[2]])[None, :]
    x0, a1p, a2p = _k_in(x, w0cat, b0cat, d1)

    s1 = _spmm_parts(a1p, rc4)
    t1 = _spmm_parts(a2p, rc4)
    y2 = _k_combine(t1[0], t1[1], a2p, d2)
    s2 = _spmm_parts(y2, rc4)

    h, sm, sq = _k_assemble(x0, a1p, s1[0], s1[1], y2, s2[0], s2[1], d1)
    ssum = sm[0]
    ssq = sq[0]
    mean = ssum / N
    var = ssq / N - mean * mean
    rstd = lax.rsqrt(var + EPS)
    scale = (gamma * rstd)[None, :]
    shift = (beta - mean * gamma * rstd)[None, :]

    w1cat = jnp.concatenate([W1[0], W1[1], W1[2]], axis=1)
    b1cat = jnp.concatenate([b1[0], b1[1], b1[2]])[None, :]
    y0, b1p, b2p = _k_bn_in(h, scale, shift, w1cat, b1cat, d1)

    u1 = _spmm_parts(b1p, rc4)
    v1 = _spmm_parts(b2p, rc4)
    y2b = _k_combine(v1[0], v1[1], b2p, d2)
    v2 = _spmm_parts(y2b, rc4)

    return _k_final(y0, b1p, u1[0], u1[1], y2b, v2[0], v2[1], d1, Wf,
                    bf[None, :])
